# baseline (speedup 1.0000x reference)
import jax
import jax.numpy as jnp
from jax.experimental import pallas as pl
from jax.experimental.pallas import tpu as pltpu

N = 10000
F = 128
NOUT = 64
B = 200
NB = N // B


def _dot(a, b):
    return jax.lax.dot_general(a, b, (((1,), (0,)), ((), ())),
                               preferred_element_type=jnp.float32)


def _softmax_col(l):
    e = jnp.exp(l - jnp.max(l))
    return e / jnp.sum(e)


def _fused_kernel(adj_ref, x_in_ref, wn1_ref, ws1_ref, b1_ref,
                  wn2_ref, ws2_ref, b2_ref, g1_ref, g2_ref, g1b_ref, g2b_ref,
                  pw_ref, pb_ref,
                  zenc_ref, pred_ref,
                  x_ref, s_ref, r_ref, z_ref, acc_ref, y_ref):
    p = pl.program_id(0)
    i = pl.program_id(1)

    @pl.when((p == 0) & (i == 0))
    def _():
        s_ref[...] = _dot(x_in_ref[...], wn1_ref[...])

    @pl.when(p == 0)
    def _():
        rows = x_in_ref[pl.ds(i * B, B), :]
        h1 = _dot(rows, ws1_ref[...]) + b1_ref[...]
        xb = jnp.maximum(_dot(adj_ref[...], s_ref[...]) + h1, 0.0)
        x_ref[pl.ds(i * B, B), :] = xb
        r_ref[pl.ds(i * B, B), :] = _dot(xb, g1_ref[...]) + g1b_ref[0, 0]
        z_ref[pl.ds(i * B, B), :] = _dot(xb, g2_ref[...]) + g2b_ref[0, 0]
        y_ref[pl.ds(i * B, B), :] = _dot(xb, wn2_ref[...])

    @pl.when((p == 1) & (i == 0))
    def _():
        r_ref[...] = _softmax_col(r_ref[...])
        z_ref[...] = _softmax_col(z_ref[...])
        s_ref[...] = r_ref[...] * y_ref[...]
        acc_ref[...] = jnp.zeros_like(acc_ref)

    @pl.when(p == 1)
    def _():
        x_blk = x_ref[pl.ds(i * B, B), :]
        r_blk = r_ref[pl.ds(i * B, B), :]
        z_blk = z_ref[pl.ds(i * B, B), :]
        h2 = _dot(x_blk * r_blk, ws2_ref[...]) + b2_ref[...]
        x2 = jnp.maximum(_dot(adj_ref[...], s_ref[...]) + h2, 0.0)
        zenc = (1.0 - z_blk) * x_blk + z_blk * x2
        zenc_ref[...] = zenc
        acc_ref[...] += jnp.sum(zenc, axis=0, keepdims=True)

    @pl.when((p == 1) & (i == NB - 1))
    def _():
        pred_ref[...] = _dot(acc_ref[...] * (1.0 / N), pw_ref[...]) + pb_ref[...]


def kernel(inputs, adj, Wn1, Ws1, b1, Wn2, Ws2, b2, g1w, g1b, g2w, g2b,
           e2pw, e2pb):
    f32 = jnp.float32
    full = lambda shape: pl.BlockSpec(shape, lambda p, i: (0,) * len(shape))

    zenc, pred = pl.pallas_call(
        _fused_kernel,
        grid=(2, NB),
        in_specs=[pl.BlockSpec((B, N), lambda p, i: (i, 0)),
                  full((N, F)), full((F, F)), full((F, F)), full((1, F)),
                  full((F, F)), full((F, F)), full((1, F)),
                  full((F, 1)), full((F, 1)), full((1, 1)), full((1, 1)),
                  full((F, NOUT)), full((1, NOUT))],
        out_specs=[pl.BlockSpec((B, F), lambda p, i: (p * i, 0)),
                   full((1, NOUT))],
        out_shape=[jax.ShapeDtypeStruct((N, F), f32),
                   jax.ShapeDtypeStruct((1, NOUT), f32)],
        scratch_shapes=[pltpu.VMEM((N, F), f32), pltpu.VMEM((N, F), f32),
                        pltpu.VMEM((N, 1), f32), pltpu.VMEM((N, 1), f32),
                        pltpu.VMEM((1, F), f32), pltpu.VMEM((N, F), f32)],
        compiler_params=pltpu.CompilerParams(
            vmem_limit_bytes=100 * 1024 * 1024),
    )(adj, inputs, Wn1, Ws1, b1.reshape(1, F), Wn2, Ws2, b2.reshape(1, F),
      g1w, g2w, g1b.reshape(1, 1), g2b.reshape(1, 1), e2pw,
      e2pb.reshape(1, NOUT))

    return (zenc, pred)


# speedup vs baseline: 1.0061x; 1.0061x over previous
import jax
import jax.numpy as jnp
from jax.experimental import pallas as pl
from jax.experimental.pallas import tpu as pltpu

N = 10000
F = 128
NOUT = 64
B = 200
NB = N // B


def _dot(a, b):
    return jax.lax.dot_general(a, b, (((1,), (0,)), ((), ())),
                               preferred_element_type=jnp.float32)


def _softmax_col(l):
    e = jnp.exp(l - jnp.max(l))
    return e / jnp.sum(e)


def _fused_kernel(adj_ref, x_in_ref, wn1_ref, ws1_ref, b1_ref,
                  wn2_ref, ws2_ref, b2_ref, g1_ref, g2_ref, g1b_ref, g2b_ref,
                  pw_ref, pb_ref,
                  zenc_ref, pred_ref,
                  x_ref, s_ref, r_ref, z_ref, acc_ref, y_ref):
    p = pl.program_id(0)
    i = pl.program_id(1)

    @pl.when((p == 0) & (i == 0))
    def _():
        s_ref[...] = _dot(x_in_ref[...], wn1_ref[...])

    @pl.when(p == 0)
    def _():
        rows = x_in_ref[pl.ds(i * B, B), :]
        h1 = _dot(rows, ws1_ref[...]) + b1_ref[...]
        xb = jnp.maximum(_dot(adj_ref[...], s_ref[...]) + h1, 0.0)
        x_ref[pl.ds(i * B, B), :] = xb
        y_ref[pl.ds(i * B, B), :] = _dot(xb, wn2_ref[...])

    @pl.when((p == 1) & (i == 0))
    def _():
        x = x_ref[...]
        r_ref[...] = _softmax_col(_dot(x, g1_ref[...]) + g1b_ref[0, 0])
        z_ref[...] = _softmax_col(_dot(x, g2_ref[...]) + g2b_ref[0, 0])
        s_ref[...] = r_ref[...] * y_ref[...]
        acc_ref[...] = jnp.zeros_like(acc_ref)

    @pl.when(p == 1)
    def _():
        x_blk = x_ref[pl.ds(i * B, B), :]
        r_blk = r_ref[pl.ds(i * B, B), :]
        z_blk = z_ref[pl.ds(i * B, B), :]
        h2 = _dot(x_blk * r_blk, ws2_ref[...]) + b2_ref[...]
        x2 = jnp.maximum(_dot(adj_ref[...], s_ref[...]) + h2, 0.0)
        zenc = (1.0 - z_blk) * x_blk + z_blk * x2
        zenc_ref[...] = zenc
        acc_ref[...] += jnp.sum(zenc, axis=0, keepdims=True)

    @pl.when((p == 1) & (i == NB - 1))
    def _():
        pred_ref[...] = _dot(acc_ref[...] * (1.0 / N), pw_ref[...]) + pb_ref[...]


def kernel(inputs, adj, Wn1, Ws1, b1, Wn2, Ws2, b2, g1w, g1b, g2w, g2b,
           e2pw, e2pb):
    f32 = jnp.float32
    full = lambda shape: pl.BlockSpec(shape, lambda p, i: (0,) * len(shape))

    zenc, pred = pl.pallas_call(
        _fused_kernel,
        grid=(2, NB),
        in_specs=[pl.BlockSpec((B, N), lambda p, i: (i, 0)),
                  full((N, F)), full((F, F)), full((F, F)), full((1, F)),
                  full((F, F)), full((F, F)), full((1, F)),
                  full((F, 1)), full((F, 1)), full((1, 1)), full((1, 1)),
                  full((F, NOUT)), full((1, NOUT))],
        out_specs=[pl.BlockSpec((B, F), lambda p, i: (p * i, 0)),
                   full((1, NOUT))],
        out_shape=[jax.ShapeDtypeStruct((N, F), f32),
                   jax.ShapeDtypeStruct((1, NOUT), f32)],
        scratch_shapes=[pltpu.VMEM((N, F), f32), pltpu.VMEM((N, F), f32),
                        pltpu.VMEM((N, 1), f32), pltpu.VMEM((N, 1), f32),
                        pltpu.VMEM((1, F), f32), pltpu.VMEM((N, F), f32)],
        compiler_params=pltpu.CompilerParams(
            vmem_limit_bytes=100 * 1024 * 1024),
    )(adj, inputs, Wn1, Ws1, b1.reshape(1, F), Wn2, Ws2, b2.reshape(1, F),
      g1w, g2w, g1b.reshape(1, 1), g2b.reshape(1, 1), e2pw,
      e2pb.reshape(1, NOUT))

    return (zenc, pred)


# speedup vs baseline: 1.0255x; 1.0193x over previous
import jax
import jax.numpy as jnp
from jax.experimental import pallas as pl
from jax.experimental.pallas import tpu as pltpu

N = 10000
F = 128
NOUT = 64
B = 200
NB = N // B


def _dot(a, b):
    return jax.lax.dot_general(a, b, (((1,), (0,)), ((), ())),
                               preferred_element_type=jnp.float32)


def _softmax_col(l):
    e = jnp.exp(l - jnp.max(l))
    return e / jnp.sum(e)


def _fused_kernel(adj_ref, x_in_ref, wn1_ref, ws1_ref, b1_ref,
                  wn2_ref, ws2_ref, b2_ref, g1_ref, g2_ref, g1b_ref, g2b_ref,
                  pw_ref, pb_ref,
                  zenc_ref, pred_ref,
                  x_ref, s_ref, r_ref, z_ref, acc_ref):
    p = pl.program_id(0)
    i = pl.program_id(1)

    @pl.when((p == 0) & (i == 0))
    def _():
        s_ref[...] = _dot(x_in_ref[...], wn1_ref[...])

    @pl.when(p == 0)
    def _():
        rows = x_in_ref[pl.ds(i * B, B), :]
        h1 = _dot(rows, ws1_ref[...]) + b1_ref[...]
        x_ref[pl.ds(i * B, B), :] = jnp.maximum(
            _dot(adj_ref[...], s_ref[...]) + h1, 0.0)

    @pl.when((p == 1) & (i == 0))
    def _():
        x = x_ref[...]
        r_ref[...] = _softmax_col(_dot(x, g1_ref[...]) + g1b_ref[0, 0])
        z_ref[...] = _softmax_col(_dot(x, g2_ref[...]) + g2b_ref[0, 0])
        s_ref[...] = _dot(x * r_ref[...], wn2_ref[...])
        acc_ref[...] = jnp.zeros_like(acc_ref)

    @pl.when(p == 1)
    def _():
        x_blk = x_ref[pl.ds(i * B, B), :]
        r_blk = r_ref[pl.ds(i * B, B), :]
        z_blk = z_ref[pl.ds(i * B, B), :]
        h2 = _dot(x_blk * r_blk, ws2_ref[...]) + b2_ref[...]
        x2 = jnp.maximum(_dot(adj_ref[...], s_ref[...]) + h2, 0.0)
        zenc = (1.0 - z_blk) * x_blk + z_blk * x2
        zenc_ref[...] = zenc
        acc_ref[...] += jnp.sum(zenc, axis=0, keepdims=True)

    @pl.when((p == 1) & (i == NB - 1))
    def _():
        pred_ref[...] = _dot(acc_ref[...] * (1.0 / N), pw_ref[...]) + pb_ref[...]


def kernel(inputs, adj, Wn1, Ws1, b1, Wn2, Ws2, b2, g1w, g1b, g2w, g2b,
           e2pw, e2pb):
    f32 = jnp.float32
    full = lambda shape: pl.BlockSpec(shape, lambda p, i: (0,) * len(shape))

    zenc, pred = pl.pallas_call(
        _fused_kernel,
        grid=(2, NB),
        in_specs=[pl.BlockSpec((B, N), lambda p, i: (i, 0)),
                  full((N, F)), full((F, F)), full((F, F)), full((1, F)),
                  full((F, F)), full((F, F)), full((1, F)),
                  full((F, 1)), full((F, 1)), full((1, 1)), full((1, 1)),
                  full((F, NOUT)), full((1, NOUT))],
        out_specs=[pl.BlockSpec((B, F), lambda p, i: (p * i, 0)),
                   full((1, NOUT))],
        out_shape=[jax.ShapeDtypeStruct((N, F), f32),
                   jax.ShapeDtypeStruct((1, NOUT), f32)],
        scratch_shapes=[pltpu.VMEM((N, F), f32), pltpu.VMEM((N, F), f32),
                        pltpu.VMEM((N, 1), f32), pltpu.VMEM((N, 1), f32),
                        pltpu.VMEM((1, F), f32)],
        compiler_params=pltpu.CompilerParams(
            vmem_limit_bytes=100 * 1024 * 1024),
    )(adj, inputs, Wn1, Ws1, b1.reshape(1, F), Wn2, Ws2, b2.reshape(1, F),
      g1w, g2w, g1b.reshape(1, 1), g2b.reshape(1, 1), e2pw,
      e2pb.reshape(1, NOUT))

    return (zenc, pred)


# speedup vs baseline: 1.0371x; 1.0112x over previous
import jax
import jax.numpy as jnp
from jax.experimental import pallas as pl
from jax.experimental.pallas import tpu as pltpu

N = 10000
F = 128
NOUT = 64
B = 400
NB = N // B


def _dot(a, b):
    return jax.lax.dot_general(a, b, (((1,), (0,)), ((), ())),
                               preferred_element_type=jnp.float32)


def _softmax_col(l):
    e = jnp.exp(l - jnp.max(l))
    return e / jnp.sum(e)


def _fused_kernel(adj_ref, x_in_ref, wn1_ref, ws1_ref, b1_ref,
                  wn2_ref, ws2_ref, b2_ref, g1_ref, g2_ref, g1b_ref, g2b_ref,
                  pw_ref, pb_ref,
                  zenc_ref, pred_ref,
                  x_ref, s_ref, g_ref, acc_ref):
    p = pl.program_id(0)
    i = pl.program_id(1)

    @pl.when((p == 0) & (i == 0))
    def _():
        s_ref[...] = _dot(x_in_ref[...], wn1_ref[...])

    @pl.when(p == 0)
    def _():
        rows = x_in_ref[pl.ds(i * B, B), :]
        h1 = _dot(rows, ws1_ref[...]) + b1_ref[...]
        x_ref[pl.ds(i * B, B), :] = jnp.maximum(
            _dot(adj_ref[...], s_ref[...]) + h1, 0.0)

    @pl.when((p == 1) & (i == 0))
    def _():
        x = x_ref[...]
        r = _softmax_col(_dot(x, g1_ref[...]) + g1b_ref[0, 0])
        z = _softmax_col(_dot(x, g2_ref[...]) + g2b_ref[0, 0])
        g_ref[...] = jnp.concatenate([r, z], axis=1)
        s_ref[...] = _dot(x * r, wn2_ref[...])
        acc_ref[...] = jnp.zeros_like(acc_ref)

    @pl.when(p == 1)
    def _():
        x_blk = x_ref[pl.ds(i * B, B), :]
        g_blk = g_ref[pl.ds(i * B, B), :]
        r_blk = g_blk[:, 0:1]
        z_blk = g_blk[:, 1:2]
        h2 = _dot(x_blk * r_blk, ws2_ref[...]) + b2_ref[...]
        x2 = jnp.maximum(_dot(adj_ref[...], s_ref[...]) + h2, 0.0)
        zenc = (1.0 - z_blk) * x_blk + z_blk * x2
        zenc_ref[...] = zenc
        acc_ref[...] += jnp.sum(zenc, axis=0, keepdims=True)

    @pl.when((p == 1) & (i == NB - 1))
    def _():
        pred_ref[...] = _dot(acc_ref[...] * (1.0 / N), pw_ref[...]) + pb_ref[...]


def kernel(inputs, adj, Wn1, Ws1, b1, Wn2, Ws2, b2, g1w, g1b, g2w, g2b,
           e2pw, e2pb):
    f32 = jnp.float32
    full = lambda shape: pl.BlockSpec(shape, lambda p, i: (0,) * len(shape))

    zenc, pred = pl.pallas_call(
        _fused_kernel,
        grid=(2, NB),
        in_specs=[pl.BlockSpec((B, N), lambda p, i: (i, 0)),
                  full((N, F)), full((F, F)), full((F, F)), full((1, F)),
                  full((F, F)), full((F, F)), full((1, F)),
                  full((F, 1)), full((F, 1)), full((1, 1)), full((1, 1)),
                  full((F, NOUT)), full((1, NOUT))],
        out_specs=[pl.BlockSpec((B, F), lambda p, i: (p * i, 0)),
                   full((1, NOUT))],
        out_shape=[jax.ShapeDtypeStruct((N, F), f32),
                   jax.ShapeDtypeStruct((1, NOUT), f32)],
        scratch_shapes=[pltpu.VMEM((N, F), f32), pltpu.VMEM((N, F), f32),
                        pltpu.VMEM((N, 2), f32),
                        pltpu.VMEM((1, F), f32)],
        compiler_params=pltpu.CompilerParams(
            vmem_limit_bytes=100 * 1024 * 1024),
    )(adj, inputs, Wn1, Ws1, b1.reshape(1, F), Wn2, Ws2, b2.reshape(1, F),
      g1w, g2w, g1b.reshape(1, 1), g2b.reshape(1, 1), e2pw,
      e2pb.reshape(1, NOUT))

    return (zenc, pred)


# speedup vs baseline: 1.0426x; 1.0053x over previous
import jax
import jax.numpy as jnp
from jax.experimental import pallas as pl
from jax.experimental.pallas import tpu as pltpu

N = 10000
F = 128
NOUT = 64
B = 200
NB = N // (2 * B)


def _dot(a, b):
    return jax.lax.dot_general(a, b, (((1,), (0,)), ((), ())),
                               preferred_element_type=jnp.float32)


def _softmax_col(l):
    e = jnp.exp(l - jnp.max(l))
    return e / jnp.sum(e)


def _fused_kernel(adja_ref, adjb_ref, x_in_ref, wn1_ref, ws1_ref, b1_ref,
                  wn2_ref, ws2_ref, b2_ref, g1_ref, g2_ref, g1b_ref, g2b_ref,
                  pw_ref, pb_ref,
                  zenc_ref, pred_ref,
                  x_ref, s_ref, g_ref, acc_ref):
    p = pl.program_id(0)
    i = pl.program_id(1)

    @pl.when((p == 0) & (i == 0))
    def _():
        s_ref[...] = _dot(x_in_ref[...], wn1_ref[...])

    @pl.when(p == 0)
    def _():
        for k, a_ref in ((0, adja_ref), (1, adjb_ref)):
            off = (2 * i + k) * B
            rows = x_in_ref[pl.ds(off, B), :]
            h1 = _dot(rows, ws1_ref[...]) + b1_ref[...]
            x_ref[pl.ds(off, B), :] = jnp.maximum(
                _dot(a_ref[...], s_ref[...]) + h1, 0.0)

    @pl.when((p == 1) & (i == 0))
    def _():
        x = x_ref[...]
        r = _softmax_col(_dot(x, g1_ref[...]) + g1b_ref[0, 0])
        z = _softmax_col(_dot(x, g2_ref[...]) + g2b_ref[0, 0])
        g_ref[...] = jnp.concatenate([r, z], axis=1)
        s_ref[...] = _dot(x * r, wn2_ref[...])
        acc_ref[...] = jnp.zeros_like(acc_ref)

    @pl.when(p == 1)
    def _():
        for k, a_ref in ((0, adja_ref), (1, adjb_ref)):
            off = (2 * i + k) * B
            x_blk = x_ref[pl.ds(off, B), :]
            g_blk = g_ref[pl.ds(off, B), :]
            r_blk = g_blk[:, 0:1]
            z_blk = g_blk[:, 1:2]
            h2 = _dot(x_blk * r_blk, ws2_ref[...]) + b2_ref[...]
            x2 = jnp.maximum(_dot(a_ref[...], s_ref[...]) + h2, 0.0)
            zenc = (1.0 - z_blk) * x_blk + z_blk * x2
            zenc_ref[pl.ds(k * B, B), :] = zenc
            acc_ref[...] += jnp.sum(zenc, axis=0, keepdims=True)

    @pl.when((p == 1) & (i == NB - 1))
    def _():
        pred_ref[...] = _dot(acc_ref[...] * (1.0 / N), pw_ref[...]) + pb_ref[...]


def kernel(inputs, adj, Wn1, Ws1, b1, Wn2, Ws2, b2, g1w, g1b, g2w, g2b,
           e2pw, e2pb):
    f32 = jnp.float32
    full = lambda shape: pl.BlockSpec(shape, lambda p, i: (0,) * len(shape))

    zenc, pred = pl.pallas_call(
        _fused_kernel,
        grid=(2, NB),
        in_specs=[pl.BlockSpec((B, N), lambda p, i: (2 * i, 0)),
                  pl.BlockSpec((B, N), lambda p, i: (2 * i + 1, 0)),
                  full((N, F)), full((F, F)), full((F, F)), full((1, F)),
                  full((F, F)), full((F, F)), full((1, F)),
                  full((F, 1)), full((F, 1)), full((1, 1)), full((1, 1)),
                  full((F, NOUT)), full((1, NOUT))],
        out_specs=[pl.BlockSpec((2 * B, F), lambda p, i: (p * i, 0)),
                   full((1, NOUT))],
        out_shape=[jax.ShapeDtypeStruct((N, F), f32),
                   jax.ShapeDtypeStruct((1, NOUT), f32)],
        scratch_shapes=[pltpu.VMEM((N, F), f32), pltpu.VMEM((N, F), f32),
                        pltpu.VMEM((N, 2), f32),
                        pltpu.VMEM((1, F), f32)],
        compiler_params=pltpu.CompilerParams(
            vmem_limit_bytes=100 * 1024 * 1024),
    )(adj, adj, inputs, Wn1, Ws1, b1.reshape(1, F), Wn2, Ws2, b2.reshape(1, F),
      g1w, g2w, g1b.reshape(1, 1), g2b.reshape(1, 1), e2pw,
      e2pb.reshape(1, NOUT))

    return (zenc, pred)


# speedup vs baseline: 1.0616x; 1.0182x over previous
import jax
import jax.numpy as jnp
from jax.experimental import pallas as pl
from jax.experimental.pallas import tpu as pltpu

N = 10000
F = 128
NOUT = 64
B = 200
NB = N // (2 * B)


def _dot(a, b):
    return jax.lax.dot_general(a, b, (((1,), (0,)), ((), ())),
                               preferred_element_type=jnp.float32)


def _softmax_col(l):
    e = jnp.exp(l - jnp.max(l))
    return e / jnp.sum(e)


def _fused_kernel(adja_ref, adjb_ref, x_in_ref, wn1_ref, ws1_ref, b1_ref,
                  wn2_ref, ws2_ref, b2_ref, g1_ref, g2_ref, g1b_ref, g2b_ref,
                  pw_ref, pb_ref,
                  zenc_ref, pred_ref,
                  x_ref, s_ref, g_ref, acc_ref, c_ref):
    p = pl.program_id(0)
    i = pl.program_id(1)

    @pl.when((p == 0) & (i == 0))
    def _():
        s_ref[...] = _dot(x_in_ref[...], wn1_ref[...])

    @pl.when(p == 0)
    def _():
        for k, a_ref in ((0, adja_ref), (1, adjb_ref)):
            off = (2 * i + k) * B
            rows = x_in_ref[pl.ds(off, B), :]
            h1 = _dot(rows, ws1_ref[...]) + b1_ref[...]
            x_ref[pl.ds(off, B), :] = jnp.maximum(
                _dot(a_ref[...], s_ref[...]) + h1, 0.0)

    @pl.when((p == 0) & (i == NB - 1))
    def _():
        for j in range(8):
            c_ref[pl.ds(j * (B // 8), B // 8), :] = (
                adjb_ref[pl.ds(j * (B // 8), B // 8), :])

    @pl.when((p == 1) & (i == 0))
    def _():
        C = N // 5
        m = jnp.full((1, 2), -jnp.inf, jnp.float32)
        for j in range(5):
            xc = x_ref[pl.ds(j * C, C), :]
            l1 = _dot(xc, g1_ref[...]) + g1b_ref[0, 0]
            l2 = _dot(xc, g2_ref[...]) + g2b_ref[0, 0]
            l = jnp.concatenate([l1, l2], axis=1)
            g_ref[pl.ds(j * C, C), :] = l
            m = jnp.maximum(m, jnp.max(l, axis=0, keepdims=True))
        s = jnp.zeros((1, 2), jnp.float32)
        for j in range(5):
            e = jnp.exp(g_ref[pl.ds(j * C, C), :] - m)
            g_ref[pl.ds(j * C, C), :] = e
            s += jnp.sum(e, axis=0, keepdims=True)
        inv = 1.0 / s
        for j in range(5):
            gch = g_ref[pl.ds(j * C, C), :] * inv
            g_ref[pl.ds(j * C, C), :] = gch
            s_ref[pl.ds(j * C, C), :] = _dot(
                x_ref[pl.ds(j * C, C), :] * gch[:, 0:1], wn2_ref[...])
        acc_ref[...] = jnp.zeros_like(acc_ref)

    def _layer2_block(k, a_ref):
        off = (2 * i + k) * B
        x_blk = x_ref[pl.ds(off, B), :]
        g_blk = g_ref[pl.ds(off, B), :]
        r_blk = g_blk[:, 0:1]
        z_blk = g_blk[:, 1:2]
        h2 = _dot(x_blk * r_blk, ws2_ref[...]) + b2_ref[...]
        x2 = jnp.maximum(_dot(a_ref[...], s_ref[...]) + h2, 0.0)
        zenc = (1.0 - z_blk) * x_blk + z_blk * x2
        zenc_ref[pl.ds(k * B, B), :] = zenc
        acc_ref[...] += jnp.sum(zenc, axis=0, keepdims=True)

    @pl.when(p == 1)
    def _():
        _layer2_block(0, adja_ref)

    @pl.when((p == 1) & (i < NB - 1))
    def _():
        _layer2_block(1, adjb_ref)

    @pl.when((p == 1) & (i == NB - 1))
    def _():
        _layer2_block(1, c_ref)

    @pl.when((p == 1) & (i == NB - 1))
    def _():
        pred_ref[...] = _dot(acc_ref[...] * (1.0 / N), pw_ref[...]) + pb_ref[...]


def kernel(inputs, adj, Wn1, Ws1, b1, Wn2, Ws2, b2, g1w, g1b, g2w, g2b,
           e2pw, e2pb):
    f32 = jnp.float32
    full = lambda shape: pl.BlockSpec(shape, lambda p, i: (0,) * len(shape))

    zenc, pred = pl.pallas_call(
        _fused_kernel,
        grid=(2, NB),
        in_specs=[pl.BlockSpec((B, N), lambda p, i: (2 * i, 0)),
                  pl.BlockSpec(
                      (B, N),
                      lambda p, i: (jnp.where((p == 1) & (i == NB - 1),
                                              2 * i - 1, 2 * i + 1), 0)),
                  full((N, F)), full((F, F)), full((F, F)), full((1, F)),
                  full((F, F)), full((F, F)), full((1, F)),
                  full((F, 1)), full((F, 1)), full((1, 1)), full((1, 1)),
                  full((F, NOUT)), full((1, NOUT))],
        out_specs=[pl.BlockSpec((2 * B, F), lambda p, i: (p * i, 0)),
                   full((1, NOUT))],
        out_shape=[jax.ShapeDtypeStruct((N, F), f32),
                   jax.ShapeDtypeStruct((1, NOUT), f32)],
        scratch_shapes=[pltpu.VMEM((N, F), f32), pltpu.VMEM((N, F), f32),
                        pltpu.VMEM((N, 2), f32),
                        pltpu.VMEM((1, F), f32), pltpu.VMEM((B, N), f32)],
        compiler_params=pltpu.CompilerParams(
            vmem_limit_bytes=100 * 1024 * 1024),
    )(adj, adj, inputs, Wn1, Ws1, b1.reshape(1, F), Wn2, Ws2, b2.reshape(1, F),
      g1w, g2w, g1b.reshape(1, 1), g2b.reshape(1, 1), e2pw,
      e2pb.reshape(1, NOUT))

    return (zenc, pred)


# speedup vs baseline: 1.0660x; 1.0042x over previous
import jax
import jax.numpy as jnp
from jax.experimental import pallas as pl
from jax.experimental.pallas import tpu as pltpu

N = 10000
F = 128
NOUT = 64
B = 200
NB = N // (2 * B)


def _dot(a, b):
    return jax.lax.dot_general(a, b, (((1,), (0,)), ((), ())),
                               preferred_element_type=jnp.float32)


def _softmax_col(l):
    e = jnp.exp(l - jnp.max(l))
    return e / jnp.sum(e)


def _fused_kernel(adja_ref, adjb_ref, x_in_ref, wn1_ref, ws1_ref, b1_ref,
                  wn2_ref, ws2_ref, b2_ref, g1_ref, g2_ref, g1b_ref, g2b_ref,
                  pw_ref, pb_ref,
                  zenc_ref, pred_ref,
                  x_ref, s_ref, g_ref, acc_ref, c_ref):
    p = pl.program_id(0)
    i = pl.program_id(1)

    @pl.when((p == 0) & (i == 0))
    def _():
        s_ref[...] = _dot(x_in_ref[...], wn1_ref[...])

    @pl.when(p == 0)
    def _():
        for k, a_ref in ((0, adja_ref), (1, adjb_ref)):
            off = (2 * i + k) * B
            rows = x_in_ref[pl.ds(off, B), :]
            h1 = _dot(rows, ws1_ref[...]) + b1_ref[...]
            x_ref[pl.ds(off, B), :] = jnp.maximum(
                _dot(a_ref[...], s_ref[...]) + h1, 0.0)

    @pl.when((p == 0) & (i == NB - 1))
    def _():
        for j in range(8):
            c_ref[pl.ds(j * (B // 8), B // 8), :] = (
                adjb_ref[pl.ds(j * (B // 8), B // 8), :])
        C = N // 5
        m = jnp.full((1, 2), -jnp.inf, jnp.float32)
        for j in range(5):
            xc = x_ref[pl.ds(j * C, C), :]
            l1 = _dot(xc, g1_ref[...]) + g1b_ref[0, 0]
            l2 = _dot(xc, g2_ref[...]) + g2b_ref[0, 0]
            l = jnp.concatenate([l1, l2], axis=1)
            g_ref[pl.ds(j * C, C), :] = l
            m = jnp.maximum(m, jnp.max(l, axis=0, keepdims=True))
        acc_ref[0:1, 0:2] = m

    @pl.when((p == 1) & (i == 0))
    def _():
        C = N // 5
        m = acc_ref[0:1, 0:2]
        s = jnp.zeros((1, 2), jnp.float32)
        for j in range(5):
            e = jnp.exp(g_ref[pl.ds(j * C, C), :] - m)
            g_ref[pl.ds(j * C, C), :] = e
            s += jnp.sum(e, axis=0, keepdims=True)
        inv = 1.0 / s
        for j in range(5):
            gch = g_ref[pl.ds(j * C, C), :] * inv
            g_ref[pl.ds(j * C, C), :] = gch
            s_ref[pl.ds(j * C, C), :] = _dot(
                x_ref[pl.ds(j * C, C), :] * gch[:, 0:1], wn2_ref[...])
        acc_ref[...] = jnp.zeros_like(acc_ref)

    def _layer2_block(k, a_ref):
        off = (2 * i + k) * B
        x_blk = x_ref[pl.ds(off, B), :]
        g_blk = g_ref[pl.ds(off, B), :]
        r_blk = g_blk[:, 0:1]
        z_blk = g_blk[:, 1:2]
        h2 = _dot(x_blk * r_blk, ws2_ref[...]) + b2_ref[...]
        x2 = jnp.maximum(_dot(a_ref[...], s_ref[...]) + h2, 0.0)
        zenc = (1.0 - z_blk) * x_blk + z_blk * x2
        zenc_ref[pl.ds(k * B, B), :] = zenc
        acc_ref[...] += jnp.sum(zenc, axis=0, keepdims=True)

    @pl.when(p == 1)
    def _():
        _layer2_block(0, adja_ref)

    @pl.when((p == 1) & (i < NB - 1))
    def _():
        _layer2_block(1, adjb_ref)

    @pl.when((p == 1) & (i == NB - 1))
    def _():
        _layer2_block(1, c_ref)

    @pl.when((p == 1) & (i == NB - 1))
    def _():
        pred_ref[...] = _dot(acc_ref[...] * (1.0 / N), pw_ref[...]) + pb_ref[...]


def kernel(inputs, adj, Wn1, Ws1, b1, Wn2, Ws2, b2, g1w, g1b, g2w, g2b,
           e2pw, e2pb):
    f32 = jnp.float32
    full = lambda shape: pl.BlockSpec(shape, lambda p, i: (0,) * len(shape))

    zenc, pred = pl.pallas_call(
        _fused_kernel,
        grid=(2, NB),
        in_specs=[pl.BlockSpec((B, N), lambda p, i: (2 * i, 0)),
                  pl.BlockSpec(
                      (B, N),
                      lambda p, i: (jnp.where((p == 1) & (i == NB - 1),
                                              2 * i - 1, 2 * i + 1), 0)),
                  full((N, F)), full((F, F)), full((F, F)), full((1, F)),
                  full((F, F)), full((F, F)), full((1, F)),
                  full((F, 1)), full((F, 1)), full((1, 1)), full((1, 1)),
                  full((F, NOUT)), full((1, NOUT))],
        out_specs=[pl.BlockSpec((2 * B, F), lambda p, i: (p * i, 0)),
                   full((1, NOUT))],
        out_shape=[jax.ShapeDtypeStruct((N, F), f32),
                   jax.ShapeDtypeStruct((1, NOUT), f32)],
        scratch_shapes=[pltpu.VMEM((N, F), f32), pltpu.VMEM((N, F), f32),
                        pltpu.VMEM((N, 2), f32),
                        pltpu.VMEM((1, F), f32), pltpu.VMEM((B, N), f32)],
        compiler_params=pltpu.CompilerParams(
            vmem_limit_bytes=100 * 1024 * 1024),
    )(adj, adj, inputs, Wn1, Ws1, b1.reshape(1, F), Wn2, Ws2, b2.reshape(1, F),
      g1w, g2w, g1b.reshape(1, 1), g2b.reshape(1, 1), e2pw,
      e2pb.reshape(1, NOUT))

    return (zenc, pred)
